# Initial kernel scaffold; baseline (speedup 1.0000x reference)
import jax
import jax.numpy as jnp
from jax.experimental import pallas as pl


def kernel(input_ids, weight):
    raise NotImplementedError("write your pallas kernel here")


# speedup vs baseline: 1.1029x; 1.1029x over previous
import functools

import jax
import jax.numpy as jnp
from jax import lax
from jax.experimental import pallas as pl
from jax.experimental.pallas import tpu as pltpu
from jax.experimental.pallas import tpu_sc as plsc

_VOCAB = 1000000
_EMBED = 32
_BATCH = 16384
_HIST = 50
_B = _BATCH * _HIST
_NC = 2
_NS = 16
_NW = _NC * _NS
_B_PER_W = _B // _NW
_CHUNK = 1600
_NCHUNK = _B_PER_W // _CHUNK

_mesh = plsc.VectorSubcoreMesh(core_axis_name="c", subcore_axis_name="s")


@functools.partial(
    pl.kernel,
    out_type=jax.ShapeDtypeStruct((_B, _EMBED), jnp.float32),
    mesh=_mesh,
    scratch_types=[
        pltpu.VMEM((_CHUNK,), jnp.int32),
        pltpu.VMEM((_CHUNK, _EMBED), jnp.float32),
        pltpu.SemaphoreType.DMA,
    ],
    compiler_params=pltpu.CompilerParams(use_tc_tiling_on_sc=False),
)
def _emb_lookup(idx_hbm, table_hbm, out_hbm, idx_v, rows_v, sem):
    wid = lax.axis_index("s") * _NC + lax.axis_index("c")
    base = wid * _B_PER_W

    @pl.loop(0, _NCHUNK)
    def _chunk(i):
        off = base + i * _CHUNK
        pltpu.sync_copy(idx_hbm.at[pl.ds(off, _CHUNK)], idx_v)
        pltpu.async_copy(table_hbm.at[idx_v], rows_v, sem).wait()
        pltpu.sync_copy(rows_v, out_hbm.at[pl.ds(off, _CHUNK)])


def kernel(input_ids, weight):
    flat_ids = input_ids.reshape(_B).astype(jnp.int32)
    out = _emb_lookup(flat_ids, weight)
    return out.reshape(_BATCH, _HIST, _EMBED)


# speedup vs baseline: 1.1126x; 1.0088x over previous
import functools

import jax
import jax.numpy as jnp
from jax import lax
from jax.experimental import pallas as pl
from jax.experimental.pallas import tpu as pltpu
from jax.experimental.pallas import tpu_sc as plsc

_VOCAB = 1000000
_EMBED = 32
_BATCH = 16384
_HIST = 50
_B = _BATCH * _HIST
_NC = 2
_NS = 16
_NW = _NC * _NS
_B_PER_W = _B // _NW
_CHUNK = 1280
_NCHUNK = _B_PER_W // _CHUNK

_mesh = plsc.VectorSubcoreMesh(core_axis_name="c", subcore_axis_name="s")


@functools.partial(
    pl.kernel,
    out_type=jax.ShapeDtypeStruct((_B, _EMBED), jnp.float32),
    mesh=_mesh,
    scratch_types=[
        pltpu.VMEM((_NCHUNK, _CHUNK), jnp.int32),
        pltpu.VMEM((2, _CHUNK, _EMBED), jnp.float32),
        pltpu.SemaphoreType.DMA,
        pltpu.SemaphoreType.DMA,
        pltpu.SemaphoreType.DMA,
        pltpu.SemaphoreType.DMA,
    ],
    compiler_params=pltpu.CompilerParams(use_tc_tiling_on_sc=False),
)
def _emb_lookup(idx_hbm, table_hbm, out_hbm, idx_v, rows_v, sg0, sg1, sw0, sw1):
    wid = lax.axis_index("s") * _NC + lax.axis_index("c")
    base = wid * _B_PER_W
    sg = (sg0, sg1)
    sw = (sw0, sw1)

    pltpu.sync_copy(idx_hbm.at[pl.ds(wid * _NCHUNK, _NCHUNK)], idx_v)

    def start_gather(i, b):
        return pltpu.async_copy(table_hbm.at[idx_v.at[i]], rows_v.at[b], sg[b])

    def start_write(i, b):
        return pltpu.async_copy(
            rows_v.at[b], out_hbm.at[pl.ds(base + i * _CHUNK, _CHUNK)], sw[b])

    g_desc = [None, None]
    w_desc = [None, None]
    g_desc[0] = start_gather(0, 0)
    for i in range(_NCHUNK):
        b = i % 2
        b2 = 1 - b
        if i + 1 < _NCHUNK:
            if w_desc[b2] is not None:
                w_desc[b2].wait()
            g_desc[b2] = start_gather(i + 1, b2)
        g_desc[b].wait()
        w_desc[b] = start_write(i, b)
    w_desc[0].wait()
    w_desc[1].wait()


def kernel(input_ids, weight):
    flat_ids = input_ids.reshape(_NW * _NCHUNK, _CHUNK).astype(jnp.int32)
    out = _emb_lookup(flat_ids, weight)
    return out.reshape(_BATCH, _HIST, _EMBED)


# speedup vs baseline: 1.4677x; 1.3192x over previous
import functools

import jax
import jax.numpy as jnp
from jax import lax
from jax.experimental import pallas as pl
from jax.experimental.pallas import tpu as pltpu
from jax.experimental.pallas import tpu_sc as plsc

_VOCAB = 1000000
_EMBED = 32
_BATCH = 16384
_HIST = 50
_NC = 2
_NS = 16
_NW = _NC * _NS
_BB = _BATCH // _NW
_NT = _BB // 16

_mesh = plsc.VectorSubcoreMesh(core_axis_name="c", subcore_axis_name="s")


@functools.partial(
    pl.kernel,
    out_type=jax.ShapeDtypeStruct((_HIST, _EMBED, _BATCH), jnp.float32),
    mesh=_mesh,
    scratch_types=[
        pltpu.VMEM((_BB, _HIST), jnp.int32),
        pltpu.VMEM((_HIST, _BB), jnp.int32),
        pltpu.VMEM((2, _BB, _EMBED), jnp.float32),
        pltpu.VMEM((2, _EMBED, _BB), jnp.float32),
        [pltpu.SemaphoreType.DMA] * 2,
        [pltpu.SemaphoreType.DMA] * 2,
    ],
    compiler_params=pltpu.CompilerParams(
        use_tc_tiling_on_sc=False, needs_layout_passes=False),
)
def _emb_lookup(ids_hbm, table_hbm, out_hbm, idsv, ids_t, rows, outv, sg, so):
    wid = lax.axis_index("s") * _NC + lax.axis_index("c")
    b0 = wid * _BB
    iota16 = lax.iota(jnp.int32, 16)

    pltpu.sync_copy(ids_hbm.at[pl.ds(b0, _BB), :], idsv)

    @pl.loop(0, _HIST)
    def _pre(h):
        hsplat = h + jnp.zeros((16,), jnp.int32)
        for t in range(_NT):
            r16 = t * 16 + iota16
            v = plsc.load_gather(idsv, [r16, hsplat])
            ids_t.at[h][pl.ds(t * 16, 16)] = v

    def gstart(h, buf):
        return pltpu.async_copy(table_hbm.at[ids_t.at[h]], rows.at[buf], sg[buf])

    def gwait(buf):
        pltpu.make_async_copy(table_hbm.at[ids_t.at[0]], rows.at[buf],
                              sg[buf]).wait()

    def ostart(h, buf):
        return pltpu.async_copy(outv.at[buf],
                                out_hbm.at[h, :, pl.ds(b0, _BB)], so[buf])

    def owait(buf):
        pltpu.make_async_copy(outv.at[buf],
                              out_hbm.at[0, :, pl.ds(b0, _BB)], so[buf]).wait()

    def transpose(buf):
        @pl.loop(0, _NT)
        def _tr(t):
            r16 = t * 16 + iota16
            for e in range(_EMBED):
                esplat = jnp.full((16,), e, jnp.int32)
                vals = plsc.load_gather(rows.at[buf], [r16, esplat])
                outv.at[buf, e][pl.ds(t * 16, 16)] = vals

    gstart(0, 0)

    @pl.loop(0, _HIST // 2)
    def _main(k):
        h0 = 2 * k
        h1 = h0 + 1

        @pl.when(k > 0)
        def _():
            owait(1)

        gstart(h1, 1)
        gwait(0)

        @pl.when(k > 0)
        def _():
            owait(0)

        transpose(0)
        ostart(h0, 0)

        @pl.when(k < _HIST // 2 - 1)
        def _():
            gstart(h0 + 2, 0)

        gwait(1)
        transpose(1)
        ostart(h1, 1)

    owait(0)
    owait(1)


def kernel(input_ids, weight):
    out_t = _emb_lookup(input_ids.astype(jnp.int32), weight)
    return jnp.transpose(out_t, (2, 0, 1))


# speedup vs baseline: 2.1843x; 1.4883x over previous
import functools

import jax
import jax.numpy as jnp
from jax import lax
from jax.experimental import pallas as pl
from jax.experimental.pallas import tpu as pltpu
from jax.experimental.pallas import tpu_sc as plsc

_VOCAB = 1000000
_EMBED = 32
_BATCH = 16384
_HIST = 50
_NC = 2
_NS = 16
_NW = _NC * _NS
_BB = _BATCH // _NW
_NT = _BB // 16

_mesh = plsc.VectorSubcoreMesh(core_axis_name="c", subcore_axis_name="s")


@functools.partial(
    pl.kernel,
    out_type=jax.ShapeDtypeStruct((_HIST, _EMBED, _BATCH), jnp.float32),
    mesh=_mesh,
    scratch_types=[
        pltpu.VMEM((_BB, _HIST), jnp.int32),
        pltpu.VMEM((_HIST, _BB), jnp.int32),
        pltpu.VMEM((2, _BB, _EMBED), jnp.float32),
        pltpu.VMEM((2, _EMBED, _BB), jnp.float32),
        [pltpu.SemaphoreType.DMA] * 2,
        [pltpu.SemaphoreType.DMA] * 2,
    ],
    compiler_params=pltpu.CompilerParams(
        use_tc_tiling_on_sc=False, needs_layout_passes=False),
)
def _emb_lookup(ids_hbm, table_hbm, out_hbm, idsv, ids_t, rows, outv, sg, so):
    wid = lax.axis_index("s") * _NC + lax.axis_index("c")
    b0 = wid * _BB
    iota16 = lax.iota(jnp.int32, 16)

    pltpu.sync_copy(ids_hbm.at[pl.ds(b0, _BB), :], idsv)

    @pl.loop(0, _HIST)
    def _pre(h):
        hsplat = h + jnp.zeros((16,), jnp.int32)
        for t in range(_NT):
            r16 = t * 16 + iota16
            v = plsc.load_gather(idsv, [r16, hsplat])
            ids_t.at[h][pl.ds(t * 16, 16)] = v

    def gstart(h, buf):
        return pltpu.async_copy(table_hbm.at[ids_t.at[h]], rows.at[buf], sg[buf])

    def gwait(buf):
        pltpu.make_async_copy(table_hbm.at[ids_t.at[0]], rows.at[buf],
                              sg[buf]).wait()

    def ostart(h, buf):
        return pltpu.async_copy(outv.at[buf],
                                out_hbm.at[h, :, pl.ds(b0, _BB)], so[buf])

    def owait(buf):
        pltpu.make_async_copy(outv.at[buf],
                              out_hbm.at[0, :, pl.ds(b0, _BB)], so[buf]).wait()

    def transpose(buf):
        @pl.loop(0, _NT)
        def _tr(t):
            r16 = t * 16 + iota16
            for e in range(_EMBED):
                cols = (iota16 + e) & (_EMBED - 1)
                vals = plsc.load_gather(rows.at[buf], [r16, cols])
                plsc.store_scatter(outv.at[buf], [cols, r16], vals)

    gstart(0, 0)

    @pl.loop(0, _HIST // 2)
    def _main(k):
        h0 = 2 * k
        h1 = h0 + 1

        @pl.when(k > 0)
        def _():
            owait(1)

        gstart(h1, 1)
        gwait(0)

        @pl.when(k > 0)
        def _():
            owait(0)

        transpose(0)
        ostart(h0, 0)

        @pl.when(k < _HIST // 2 - 1)
        def _():
            gstart(h0 + 2, 0)

        gwait(1)
        transpose(1)
        ostart(h1, 1)

    owait(0)
    owait(1)


def kernel(input_ids, weight):
    out_t = _emb_lookup(input_ids.astype(jnp.int32), weight)
    return jnp.transpose(out_t, (2, 0, 1))


# speedup vs baseline: 2.5001x; 1.1446x over previous
import functools

import jax
import jax.numpy as jnp
from jax import lax
from jax.experimental import pallas as pl
from jax.experimental.pallas import tpu as pltpu
from jax.experimental.pallas import tpu_sc as plsc

_VOCAB = 1000000
_EMBED = 32
_BATCH = 16384
_HIST = 50
_NC = 2
_NS = 16
_NW = _NC * _NS
_BB = _BATCH // _NW
_NT = _BB // 16

_mesh = plsc.VectorSubcoreMesh(core_axis_name="c", subcore_axis_name="s")


_EB = _EMBED // 8
_CB = _BB // 128


@functools.partial(
    pl.kernel,
    out_type=jax.ShapeDtypeStruct((_HIST, _EB, _BATCH // 128, 8, 128),
                                  jnp.float32),
    mesh=_mesh,
    scratch_types=[
        pltpu.VMEM((_BB, _HIST), jnp.int32),
        pltpu.VMEM((_HIST, _BB), jnp.int32),
        pltpu.VMEM((2, _BB, _EMBED), jnp.float32),
        pltpu.VMEM((2, _EB, _CB, 8, 128), jnp.float32),
        [pltpu.SemaphoreType.DMA] * 2,
        [pltpu.SemaphoreType.DMA] * 2,
    ],
    compiler_params=pltpu.CompilerParams(
        use_tc_tiling_on_sc=False, needs_layout_passes=False),
)
def _emb_lookup(ids_hbm, table_hbm, out_hbm, idsv, ids_t, rows, outv, sg, so):
    wid = lax.axis_index("s") * _NC + lax.axis_index("c")
    b0 = wid * _BB
    iota16 = lax.iota(jnp.int32, 16)

    pltpu.sync_copy(ids_hbm.at[pl.ds(b0, _BB), :], idsv)

    @pl.loop(0, _HIST)
    def _pre(h):
        hsplat = h + jnp.zeros((16,), jnp.int32)
        for t in range(_NT):
            r16 = t * 16 + iota16
            v = plsc.load_gather(idsv, [r16, hsplat])
            ids_t.at[h][pl.ds(t * 16, 16)] = v

    def gstart(h, buf):
        return pltpu.async_copy(table_hbm.at[ids_t.at[h]], rows.at[buf], sg[buf])

    def gwait(buf):
        pltpu.make_async_copy(table_hbm.at[ids_t.at[0]], rows.at[buf],
                              sg[buf]).wait()

    cb0 = wid * _CB

    def ostart(h, buf):
        return pltpu.async_copy(
            outv.at[buf], out_hbm.at[h, :, pl.ds(cb0, _CB), :, :], so[buf])

    def owait(buf):
        pltpu.make_async_copy(
            outv.at[buf], out_hbm.at[0, :, pl.ds(cb0, _CB), :, :],
            so[buf]).wait()

    def transpose(buf):
        @pl.loop(0, _NT)
        def _tr(t):
            r16 = t * 16 + iota16
            cbv = r16 >> 7
            cv = r16 & 127
            for e in range(_EMBED):
                cols = (iota16 + e) & (_EMBED - 1)
                vals = plsc.load_gather(rows.at[buf], [r16, cols])
                plsc.store_scatter(outv.at[buf],
                                   [cols >> 3, cbv, cols & 7, cv], vals)

    gstart(0, 0)

    @pl.loop(0, _HIST // 2)
    def _main(k):
        h0 = 2 * k
        h1 = h0 + 1

        @pl.when(k > 0)
        def _():
            owait(1)

        gstart(h1, 1)
        gwait(0)

        @pl.when(k > 0)
        def _():
            owait(0)

        transpose(0)
        ostart(h0, 0)

        @pl.when(k < _HIST // 2 - 1)
        def _():
            gstart(h0 + 2, 0)

        gwait(1)
        transpose(1)
        ostart(h1, 1)

    owait(0)
    owait(1)


def kernel(input_ids, weight):
    out5 = _emb_lookup(input_ids.astype(jnp.int32), weight)
    out_t = jnp.transpose(out5, (0, 1, 3, 2, 4)).reshape(
        _HIST, _EMBED, _BATCH)
    return jnp.transpose(out_t, (2, 0, 1))
